# baseline (speedup 1.0000x reference)
import jax
import jax.numpy as jnp
from jax import lax
from jax.experimental import pallas as pl
from jax.experimental.pallas import tpu as pltpu

NUM_TOKENS = 4096
MODEL_DIM = 1024
NUM_EXPERTS = 64
CAPACITY = 2 * NUM_TOKENS // NUM_EXPERTS
TBLK = 128
NB = NUM_TOKENS // TBLK
CHUNK = 128


def _cumsum_tokens(m):
    nb = NUM_TOKENS // CHUNK
    mb = m.reshape(nb, CHUNK, NUM_EXPERTS)
    ii = lax.broadcasted_iota(jnp.int32, (CHUNK, CHUNK), 0)
    jj = lax.broadcasted_iota(jnp.int32, (CHUNK, CHUNK), 1)
    tri = (jj <= ii).astype(jnp.float32)
    trib = jnp.broadcast_to(tri, (nb, CHUNK, CHUNK))
    within = lax.dot_general(
        trib, mb, (((2,), (1,)), ((0,), (0,))),
        preferred_element_type=jnp.float32)
    sums = within[:, CHUNK - 1, :]
    bi = lax.broadcasted_iota(jnp.int32, (nb, nb), 0)
    bj = lax.broadcasted_iota(jnp.int32, (nb, nb), 1)
    tri_x = (bj < bi).astype(jnp.float32)
    carry = lax.dot_general(
        tri_x, sums, (((1,), (0,)), ((), ())),
        preferred_element_type=jnp.float32)
    return (within + carry[:, None, :]).reshape(NUM_TOKENS, NUM_EXPERTS)


def _gate_kernel(x_ref, w_ref, noise_ref, laux_ref, cw_ref, disp_ref,
                 cwe_scr, loce_scr):
    i = pl.program_id(0)

    @pl.when(i == 0)
    def _():
        x = x_ref[...]
        w = w_ref[...]
        logits = lax.dot_general(
            x, w, (((1,), (1,)), ((), ())),
            preferred_element_type=jnp.float32)
        m = jnp.max(logits, axis=1, keepdims=True)
        ex = jnp.exp(logits - m)
        gates = ex / jnp.sum(ex, axis=1, keepdims=True)
        iota_e = lax.broadcasted_iota(
            jnp.int32, (NUM_TOKENS, NUM_EXPERTS), 1)
        e1 = jnp.argmax(logits, axis=1)
        oh1 = iota_e == e1[:, None]
        g1 = jnp.max(gates, axis=1)
        lx = jnp.where(oh1, -jnp.inf, logits + noise_ref[...])
        e2 = jnp.argmax(lx, axis=1)
        oh2 = iota_e == e2[:, None]
        g2 = jnp.sum(jnp.where(oh2, gates, 0.0), axis=1)

        cs1 = _cumsum_tokens(oh1.astype(jnp.float32))
        cs2 = _cumsum_tokens(oh2.astype(jnp.float32))
        counts1 = cs1[NUM_TOKENS - 1:NUM_TOKENS, :]
        loc1 = jnp.sum(jnp.where(oh1, cs1 - 1.0, 0.0), axis=1)
        loc2 = jnp.sum(jnp.where(oh2, cs2 - 1.0 + counts1, 0.0), axis=1)

        g1s = jnp.where(loc1 < CAPACITY, g1, 0.0)
        g2s = jnp.where(loc2 < CAPACITY, g2, 0.0)
        denom = jnp.maximum(g1s + g2s, jnp.finfo(jnp.float32).eps)
        g1n = g1s / denom
        g2n = g2s / denom

        cwe_scr[...] = (jnp.where(oh1, g1n[:, None], 0.0)
                        + jnp.where(oh2, g2n[:, None], 0.0))
        loce_scr[...] = (jnp.where(oh1, loc1[:, None], 0.0)
                         + jnp.where(oh2, loc2[:, None], 0.0)
                         ).astype(jnp.int32)

        me_sum = jnp.sum(gates, axis=0, keepdims=True)
        laux_ref[...] = jnp.sum(me_sum * counts1, axis=1, keepdims=True) / (
            float(NUM_EXPERTS) * NUM_TOKENS * NUM_TOKENS)

    cw = cwe_scr[pl.ds(i * TBLK, TBLK), :]
    loce = loce_scr[pl.ds(i * TBLK, TBLK), :]
    iota_c = lax.broadcasted_iota(
        jnp.int32, (TBLK, NUM_EXPERTS, CAPACITY), 2)
    out = jnp.where(iota_c == loce[:, :, None], cw[:, :, None], 0.0)
    cw_ref[...] = out
    disp_ref[...] = out != 0.0


def kernel(input, W):
    noise = jax.random.gumbel(
        jax.random.key(42), (NUM_TOKENS, NUM_EXPERTS), dtype=jnp.float32)
    laux, cw, disp = pl.pallas_call(
        _gate_kernel,
        grid=(NB,),
        in_specs=[
            pl.BlockSpec((NUM_TOKENS, MODEL_DIM), lambda i: (0, 0)),
            pl.BlockSpec((NUM_EXPERTS, MODEL_DIM), lambda i: (0, 0)),
            pl.BlockSpec((NUM_TOKENS, NUM_EXPERTS), lambda i: (0, 0)),
        ],
        out_specs=[
            pl.BlockSpec((1, 1), lambda i: (0, 0)),
            pl.BlockSpec((TBLK, NUM_EXPERTS, CAPACITY), lambda i: (i, 0, 0)),
            pl.BlockSpec((TBLK, NUM_EXPERTS, CAPACITY), lambda i: (i, 0, 0)),
        ],
        out_shape=[
            jax.ShapeDtypeStruct((1, 1), jnp.float32),
            jax.ShapeDtypeStruct((NUM_TOKENS, NUM_EXPERTS, CAPACITY),
                                 jnp.float32),
            jax.ShapeDtypeStruct((NUM_TOKENS, NUM_EXPERTS, CAPACITY),
                                 jnp.bool_),
        ],
        scratch_shapes=[
            pltpu.VMEM((NUM_TOKENS, NUM_EXPERTS), jnp.float32),
            pltpu.VMEM((NUM_TOKENS, NUM_EXPERTS), jnp.int32),
        ],
    )(input, W, noise)
    return laux.reshape(()), cw, disp
